# baseline (speedup 1.0000x reference)
import jax
import jax.numpy as jnp
from jax import lax
from jax.experimental import pallas as pl
from jax.experimental.pallas import tpu as pltpu
from jax.experimental.pallas import tpu_sc as plsc

_T = 256
_NC, _NS, _L = 2, 16, 16
_NW = _NC * _NS

_THRESHOLDS = (12, 16, 23, 32, 46, 64, 91)


def _dbias_sc_kernel(table_hbm, out_hbm, idx_v, buf, sem, H, D, per_w):
    w = lax.axis_index("s") * _NC + lax.axis_index("c")
    nchunk = per_w // _L
    nrow = per_w // 128
    ji = jnp.arange(_L, dtype=jnp.int32)

    def body(t, carry):
        base = w * per_w + t * _L
        h = base // D
        d = (base - h * D - (D // 2 - 1)) + ji
        a = jnp.abs(d)
        big = jnp.full((_L,), 8, jnp.int32)
        for thr in _THRESHOLDS:
            big = big + jnp.where(a >= thr, 1, 0)
        mag = jnp.where(a < 8, a, jnp.minimum(big, 15))
        b = mag + jnp.where(d > 0, 16, 0)
        idx_v[pl.ds(t * _L, _L)] = b * H + jnp.full((_L,), h, jnp.int32)
        return carry

    lax.fori_loop(0, nchunk, body, 0)

    pltpu.make_async_copy(table_hbm.at[idx_v], buf, sem).start()
    pltpu.make_async_copy(table_hbm.at[idx_v], buf, sem).wait()
    pltpu.sync_copy(buf, out_hbm.at[pl.ds(w * per_w, per_w)])


def _dbias(bias_table, D):
    NB, H = bias_table.shape
    per_w = H * D // _NW
    fn = pl.kernel(
        lambda t, o, iv, bf, sm: _dbias_sc_kernel(t, o, iv, bf, sm, H, D, per_w),
        out_type=jax.ShapeDtypeStruct((H * D,), jnp.float32),
        mesh=plsc.VectorSubcoreMesh(core_axis_name="c", subcore_axis_name="s"),
        scratch_types=[
            pltpu.VMEM((per_w,), jnp.int32),
            pltpu.VMEM((per_w,), jnp.float32),
            pltpu.SemaphoreType.DMA,
        ],
    )
    return fn(bias_table.reshape(-1)).reshape(H, D)


def _bias_tc_kernel(dbias_ref, out_ref, ds_ref, mega_ref, sems):
    h = pl.program_id(0)
    bi = pl.program_id(1)
    H = pl.num_programs(0)
    S = out_ref.shape[3]
    nb = S // _T
    W = (nb + 7) * _T
    D = ds_ref.shape[2]
    K = D // 2 - 1
    par = lax.rem(h, 3)

    def _stripe_copy(hh, row, parity):
        return pltpu.make_async_copy(
            mega_ref.at[parity, :, pl.ds((nb - 1 - row) * _T, S)],
            out_ref.at[0, hh, pl.ds(row * _T, _T), :],
            sems.at[lax.rem(hh, 2), row],
        )

    @pl.when(h > 1)
    def _wait_prev():
        _stripe_copy(h - 2, bi, lax.rem(h - 2, 3)).wait()

    @pl.when(jnp.logical_and(h == 0, bi == 0))
    def _build_ds():
        for hh in range(ds_ref.shape[0]):
            for s in range(8):
                ds_ref[hh, s, s:D] = dbias_ref[hh, 0:D - s]

    @pl.when(bi == 0)
    def _build_mega():
        c0 = K - (W - S)
        for q in range(_T // 8):
            mega_ref[par, 8 * q:8 * q + 8, :] = (
                ds_ref[h, :, c0 - 8 * q:c0 - 8 * q + W])

    _stripe_copy(h, bi, par).start()

    @pl.when(jnp.logical_and(h == H - 1, bi == nb - 1))
    def _final_wait():
        for row in range(nb):
            _stripe_copy(h - 1, row, lax.rem(h - 1, 3)).wait()
        for row in range(nb):
            _stripe_copy(h, row, par).wait()


def kernel(input_ids, bboxes, bias_table):
    B, S = input_ids.shape
    H = bias_table.shape[1]
    nb = S // _T
    D = 2 * S
    dbias = _dbias(bias_table, D)
    out = pl.pallas_call(
        _bias_tc_kernel,
        grid=(H, nb),
        in_specs=[pl.BlockSpec(memory_space=pltpu.VMEM)],
        out_specs=pl.BlockSpec(memory_space=pl.ANY),
        out_shape=jax.ShapeDtypeStruct((B, H, S, S), jnp.float32),
        scratch_shapes=[
            pltpu.VMEM((H, 8, D), jnp.float32),
            pltpu.VMEM((3, _T, (nb + 7) * _T), jnp.float32),
            pltpu.SemaphoreType.DMA((2, nb)),
        ],
        compiler_params=pltpu.CompilerParams(
            dimension_semantics=("arbitrary", "arbitrary"),
        ),
    )(dbias)
    return out


# speedup vs baseline: 1.2432x; 1.2432x over previous
import jax
import jax.numpy as jnp
from jax import lax
from jax.experimental import pallas as pl
from jax.experimental.pallas import tpu as pltpu
from jax.experimental.pallas import tpu_sc as plsc

_T = 256
_NC, _NS, _L = 2, 16, 16
_NW = _NC * _NS

_THRESHOLDS = (12, 16, 23, 32, 46, 64, 91)


def _dbias_sc_kernel(table_hbm, out_hbm, idx_v, buf, sem, D, per_w):
    w = lax.axis_index("s") * _NC + lax.axis_index("c")
    nchunk = per_w // _L
    ji = jnp.arange(_L, dtype=jnp.int32)

    def body(t, carry):
        d = (w * per_w + t * _L - (D // 2 - 1)) + ji
        a = jnp.abs(d)
        big = jnp.full((_L,), 8, jnp.int32)
        for thr in _THRESHOLDS:
            big = big + jnp.where(a >= thr, 1, 0)
        mag = jnp.where(a < 8, a, jnp.minimum(big, 15))
        idx_v[pl.ds(t * _L, _L)] = mag + jnp.where(d > 0, 16, 0)
        return carry

    lax.fori_loop(0, nchunk, body, 0)

    pltpu.make_async_copy(table_hbm.at[idx_v], buf, sem).start()
    pltpu.make_async_copy(table_hbm.at[idx_v], buf, sem).wait()
    pltpu.sync_copy(buf, out_hbm.at[pl.ds(w * per_w, per_w), :])


def _dbias(bias_table, D):
    NB, H = bias_table.shape
    per_w = D // _NW
    table_pad = jnp.zeros((NB, 128), jnp.float32).at[:, :H].set(bias_table)
    fn = pl.kernel(
        lambda t, o, iv, bf, sm: _dbias_sc_kernel(t, o, iv, bf, sm, D, per_w),
        out_type=jax.ShapeDtypeStruct((D, 128), jnp.float32),
        mesh=plsc.VectorSubcoreMesh(core_axis_name="c", subcore_axis_name="s"),
        scratch_types=[
            pltpu.VMEM((per_w,), jnp.int32),
            pltpu.VMEM((per_w, 128), jnp.float32),
            pltpu.SemaphoreType.DMA,
        ],
    )
    return fn(table_pad)[:, :H].T


def _bias_tc_kernel(dbias_ref, out_ref, ds_ref, mega_ref, sems):
    h = pl.program_id(0)
    bi = pl.program_id(1)
    H = pl.num_programs(0)
    S = out_ref.shape[3]
    nb = S // _T
    W = (nb + 7) * _T
    D = ds_ref.shape[2]
    K = D // 2 - 1
    par = lax.rem(h, 3)

    def _stripe_copy(hh, row, parity):
        return pltpu.make_async_copy(
            mega_ref.at[parity, :, pl.ds((nb - 1 - row) * _T, S)],
            out_ref.at[0, hh, pl.ds(row * _T, _T), :],
            sems.at[lax.rem(hh, 2), row],
        )

    @pl.when(h > 1)
    def _wait_prev():
        _stripe_copy(h - 2, bi, lax.rem(h - 2, 3)).wait()

    @pl.when(jnp.logical_and(h == 0, bi == 0))
    def _build_ds():
        for hh in range(ds_ref.shape[0]):
            for s in range(8):
                ds_ref[hh, s, s:D] = dbias_ref[hh, 0:D - s]

    @pl.when(bi == 0)
    def _build_mega():
        c0 = K - (W - S)
        for q in range(_T // 8):
            mega_ref[par, 8 * q:8 * q + 8, :] = (
                ds_ref[h, :, c0 - 8 * q:c0 - 8 * q + W])

    _stripe_copy(h, bi, par).start()

    @pl.when(jnp.logical_and(h == H - 1, bi == nb - 1))
    def _final_wait():
        for row in range(nb):
            _stripe_copy(h - 1, row, lax.rem(h - 1, 3)).wait()
        for row in range(nb):
            _stripe_copy(h, row, par).wait()


def kernel(input_ids, bboxes, bias_table):
    B, S = input_ids.shape
    H = bias_table.shape[1]
    nb = S // _T
    D = 2 * S
    dbias = _dbias(bias_table, D)
    out = pl.pallas_call(
        _bias_tc_kernel,
        grid=(H, nb),
        in_specs=[pl.BlockSpec(memory_space=pltpu.VMEM)],
        out_specs=pl.BlockSpec(memory_space=pl.ANY),
        out_shape=jax.ShapeDtypeStruct((B, H, S, S), jnp.float32),
        scratch_shapes=[
            pltpu.VMEM((H, 8, D), jnp.float32),
            pltpu.VMEM((3, _T, (nb + 7) * _T), jnp.float32),
            pltpu.SemaphoreType.DMA((2, nb)),
        ],
        compiler_params=pltpu.CompilerParams(
            dimension_semantics=("arbitrary", "arbitrary"),
        ),
    )(dbias)
    return out


# speedup vs baseline: 1.8801x; 1.5123x over previous
import jax
import jax.numpy as jnp
import numpy as np
from jax.experimental import pallas as pl
from jax.experimental.pallas import tpu as pltpu

_T = 512


def _bias_kernel(table_ref, out_ref, mega_ref, sems):
    h = pl.program_id(0)
    bi = pl.program_id(1)
    H = pl.num_programs(0)
    S = out_ref.shape[3]
    nb = S // _T
    nm = 2 * nb - 1
    par = jax.lax.rem(h, 3)

    def _stripe_copy(hh, row, parity):
        return pltpu.make_async_copy(
            mega_ref.at[parity, :, pl.ds((nb - 1 - row) * _T, S)],
            out_ref.at[0, hh, pl.ds(row * _T, _T), :],
            sems.at[jax.lax.rem(hh, 2), row],
        )

    @pl.when(h > 1)
    def _wait_prev():
        _stripe_copy(h - 2, bi, jax.lax.rem(h - 2, 3)).wait()

    @pl.when(bi == 0)
    def _build_mega():
        lo = nb - 2
        mega_ref[par, :, 0:lo * _T] = jnp.full(
            (_T, lo * _T), table_ref[15, h], jnp.float32)
        mega_ref[par, :, (lo + 3) * _T:nm * _T] = jnp.full(
            (_T, (nm - lo - 3) * _T), table_ref[31, h], jnp.float32)
        r = jax.lax.broadcasted_iota(jnp.int32, (_T, _T), 0)
        c = jax.lax.broadcasted_iota(jnp.int32, (_T, _T), 1)
        base = c - r
        for m, koff in ((lo, -_T), (lo + 1, 0), (lo + 2, _T)):
            d = base + koff
            rp = jnp.abs(d)
            is_small = rp < 8
            rp_safe = jnp.maximum(rp, 1).astype(jnp.float32)
            if_large = 8 + (
                jnp.log(rp_safe / 8) / np.log(128 / 8) * (16 - 8)
            ).astype(jnp.int32)
            if_large = jnp.minimum(if_large, 15)
            mag = jnp.where(is_small, rp, if_large)
            b = mag + jnp.where(d > 0, 16, 0)
            acc = jnp.full((_T, _T), table_ref[0, h], jnp.float32)
            for bb in range(1, 32):
                acc = jnp.where(b == bb, table_ref[bb, h], acc)
            mega_ref[par, :, m * _T:(m + 1) * _T] = acc

    _stripe_copy(h, bi, par).start()

    @pl.when(jnp.logical_and(h == H - 1, bi == nb - 1))
    def _final_wait():
        for row in range(nb):
            _stripe_copy(h - 1, row, jax.lax.rem(h - 1, 3)).wait()
        for row in range(nb):
            _stripe_copy(h, row, par).wait()


def kernel(input_ids, bboxes, bias_table):
    B, S = input_ids.shape
    H = bias_table.shape[1]
    nb = S // _T
    out = pl.pallas_call(
        _bias_kernel,
        grid=(H, nb),
        in_specs=[pl.BlockSpec(memory_space=pltpu.SMEM)],
        out_specs=pl.BlockSpec(memory_space=pl.ANY),
        out_shape=jax.ShapeDtypeStruct((B, H, S, S), jnp.float32),
        scratch_shapes=[
            pltpu.VMEM((3, _T, (2 * nb - 1) * _T), jnp.float32),
            pltpu.SemaphoreType.DMA((2, nb)),
        ],
        compiler_params=pltpu.CompilerParams(
            dimension_semantics=("arbitrary", "arbitrary"),
        ),
    )(bias_table)
    return out


# speedup vs baseline: 3.1675x; 1.6848x over previous
import jax
import jax.numpy as jnp
import numpy as np
from jax.experimental import pallas as pl
from jax.experimental.pallas import tpu as pltpu

_T = 128


def _bias_kernel(table_ref, out_ref, mega_ref, sems):
    h = pl.program_id(0)
    bi = pl.program_id(1)
    H = pl.num_programs(0)
    S = out_ref.shape[3]
    nb = S // _T
    nm = 2 * nb - 1
    par = jax.lax.rem(h, 3)

    def _stripe_copy(hh, row, parity):
        return pltpu.make_async_copy(
            mega_ref.at[parity, :, pl.ds((nb - 1 - row) * _T, S)],
            out_ref.at[0, hh, pl.ds(row * _T, _T), :],
            sems.at[jax.lax.rem(hh, 2), row],
        )

    @pl.when(h > 1)
    def _wait_prev():
        _stripe_copy(h - 2, bi, jax.lax.rem(h - 2, 3)).wait()

    @pl.when(bi == 0)
    def _build_mega():
        lo = nb - 2
        mega_ref[par, :, 0:lo * _T] = jnp.full(
            (_T, lo * _T), table_ref[15, h], jnp.float32)
        mega_ref[par, :, (lo + 3) * _T:nm * _T] = jnp.full(
            (_T, (nm - lo - 3) * _T), table_ref[31, h], jnp.float32)
        r = jax.lax.broadcasted_iota(jnp.int32, (_T, _T), 0)
        c = jax.lax.broadcasted_iota(jnp.int32, (_T, _T), 1)
        base = c - r
        for m, koff in ((lo, -_T), (lo + 1, 0), (lo + 2, _T)):
            d = base + koff
            rp = jnp.abs(d)
            is_small = rp < 8
            rp_safe = jnp.maximum(rp, 1).astype(jnp.float32)
            if_large = 8 + (
                jnp.log(rp_safe / 8) / np.log(128 / 8) * (16 - 8)
            ).astype(jnp.int32)
            if_large = jnp.minimum(if_large, 15)
            mag = jnp.where(is_small, rp, if_large)
            b = mag + jnp.where(d > 0, 16, 0)
            acc = jnp.full((_T, _T), table_ref[0, h], jnp.float32)
            for bb in range(1, 32):
                acc = jnp.where(b == bb, table_ref[bb, h], acc)
            mega_ref[par, :, m * _T:(m + 1) * _T] = acc

    _stripe_copy(h, bi, par).start()

    @pl.when(jnp.logical_and(h == H - 1, bi == nb - 1))
    def _final_wait():
        for row in range(nb):
            _stripe_copy(h - 1, row, jax.lax.rem(h - 1, 3)).wait()
        for row in range(nb):
            _stripe_copy(h, row, par).wait()


def kernel(input_ids, bboxes, bias_table):
    B, S = input_ids.shape
    H = bias_table.shape[1]
    nb = S // _T
    out = pl.pallas_call(
        _bias_kernel,
        grid=(H, nb),
        in_specs=[pl.BlockSpec(memory_space=pltpu.SMEM)],
        out_specs=pl.BlockSpec(memory_space=pl.ANY),
        out_shape=jax.ShapeDtypeStruct((B, H, S, S), jnp.float32),
        scratch_shapes=[
            pltpu.VMEM((3, _T, (2 * nb - 1) * _T), jnp.float32),
            pltpu.SemaphoreType.DMA((2, nb)),
        ],
        compiler_params=pltpu.CompilerParams(
            dimension_semantics=("arbitrary", "arbitrary"),
        ),
    )(bias_table)
    return out
